# Initial kernel scaffold; baseline (speedup 1.0000x reference)
import jax
import jax.numpy as jnp
from jax.experimental import pallas as pl


def kernel(x, e, W):
    raise NotImplementedError("write your pallas kernel here")


# speedup vs baseline: 1.1427x; 1.1427x over previous
import functools

import jax
import jax.numpy as jnp
from jax import lax
from jax.experimental import pallas as pl
from jax.experimental.pallas import tpu as pltpu
from jax.experimental.pallas import tpu_sc as plsc

_N = 65536
_D = 32
_K = 8192
_BN = 256
_NBLK = _N // _BN
_LOSS_SCALE = 1.25 / (_N * _D)

_NC = 2
_NS = 16
_NW = _NC * _NS
_BPW = _N // _NW
_CH = 128
_NCH = _BPW // _CH


def _argmin_body(e_ref, w_ref, idx_ref, acc_ref):
    i = pl.program_id(0)
    e_blk = e_ref[...]
    w_blk = w_ref[...]
    esq = jnp.sum(e_blk * e_blk, axis=1, keepdims=True)
    wsq = jnp.sum(w_blk * w_blk, axis=1)
    mm = lax.dot_general(e_blk, w_blk, (((1,), (1,)), ((), ())),
                         preferred_element_type=jnp.float32)
    dist = (esq + wsq[None, :]) - 2.0 * mm
    idx_ref[0, 0, :] = jnp.argmin(dist, axis=1).astype(jnp.int32)
    mv = jnp.min(dist, axis=1)

    @pl.when(i == 0)
    def _init():
        acc_ref[...] = jnp.zeros((1, 1), jnp.float32)

    acc_ref[...] = acc_ref[...] + jnp.sum(mv)

    @pl.when(i == _NBLK - 1)
    def _finish():
        acc_ref[...] = acc_ref[...] * _LOSS_SCALE


def _tc_argmin(e, w):
    return pl.pallas_call(
        _argmin_body,
        grid=(_NBLK,),
        in_specs=[
            pl.BlockSpec((_BN, _D), lambda i: (i, 0)),
            pl.BlockSpec((_K, _D), lambda i: (0, 0)),
        ],
        out_specs=[
            pl.BlockSpec((1, 1, _BN), lambda i: (i, 0, 0)),
            pl.BlockSpec((1, 1), lambda i: (0, 0)),
        ],
        out_shape=[
            jax.ShapeDtypeStruct((_NBLK, 1, _BN), jnp.int32),
            jax.ShapeDtypeStruct((1, 1), jnp.float32),
        ],
    )(e, w)


def _gather_body(w_hbm, idx_hbm, out_hbm, idx_v, rows_v, sem):
    wid = lax.axis_index("s") * _NC + lax.axis_index("c")
    pltpu.sync_copy(idx_hbm.at[pl.ds(wid * _NCH, _NCH)], idx_v)
    copies = []
    for j in range(_NCH):
        copies.append(
            pltpu.async_copy(w_hbm.at[idx_v.at[j]],
                             rows_v.at[pl.ds(j * _CH, _CH)], sem))
    for c in copies:
        c.wait()
    pltpu.sync_copy(rows_v, out_hbm.at[pl.ds(wid * _BPW, _BPW)])


@functools.partial(jax.jit, static_argnums=())
def _sc_gather(w, idx2):
    mesh = plsc.VectorSubcoreMesh(core_axis_name="c", subcore_axis_name="s")
    f = pl.kernel(
        _gather_body,
        mesh=mesh,
        compiler_params=pltpu.CompilerParams(use_tc_tiling_on_sc=False),
        out_type=jax.ShapeDtypeStruct((_N, _D), jnp.float32),
        scratch_types=[
            pltpu.VMEM((_NCH, _CH), jnp.int32),
            pltpu.VMEM((_BPW, _D), jnp.float32),
            pltpu.SemaphoreType.DMA,
        ],
    )
    return f(w, idx2)


def kernel(x, e, W):
    del x
    idx3, acc = _tc_argmin(e, W)
    idx2 = idx3.reshape(_N // _CH, _CH)
    quantized = _sc_gather(W, idx2)
    loss = acc[0, 0]
    return quantized, loss
